# baseline (speedup 1.0000x reference)
import functools

import jax
import jax.numpy as jnp
from jax import lax
from jax.experimental import pallas as pl
from jax.experimental.pallas import tpu as pltpu
from jax.experimental.pallas import tpu_sc as plsc

_MARGIN = 5.0
_ZD = 64
_VOCAB = 100
_VPAD = 128
_B = 4096
_W = 50
_NC = 2
_NS = 16
_NW = _NC * _NS
_BPW = _B // _NW
_L = 16
_GRID = 4


def _scores_body(mu_ref, sq_ref, mtT_ref, lsT_ref, out_ref):
    mu = mu_ref[...]
    sq = sq_ref[...]
    mtT = mtT_ref[...]
    ls = lsT_ref[...]
    r = jnp.exp(-ls)
    n = jnp.sum(mtT * mtT, axis=0, keepdims=True)
    m = jnp.sum(mu * mu, axis=1, keepdims=True)
    d = jnp.dot(mu, mtT, preferred_element_type=jnp.float32)
    h = 32.0 * ls + (32.0 * sq + 0.5 * (m + n) - d) * r
    out_ref[...] = h - 32.0 * jnp.log(sq) - 32.0


def _sc_hinge_body(h_hbm, pk_hbm, out_hbm, h_v, pk_v, out_v):
    wid = lax.axis_index("s") * _NC + lax.axis_index("c")
    base = wid * _BPW
    pltpu.sync_copy(h_hbm.at[pl.ds(base, _BPW)], h_v)
    pltpu.sync_copy(pk_hbm.at[:, pl.ds(base, _BPW)], pk_v)
    lane = lax.iota(jnp.int32, _L)
    ngrp = _BPW // _L
    brows = [g * _L + lane for g in range(ngrp)]

    def body(w, accs):
        new = []
        for g in range(ngrp):
            pk = pk_v[w, pl.ds(g * _L, _L)]
            pv = pk & 127
            nv = lax.shift_right_logical(pk, 7)
            gp = plsc.load_gather(h_v, [brows[g], pv])
            gn = plsc.load_gather(h_v, [brows[g], nv])
            new.append(accs[g] + jnp.maximum(0.0, _MARGIN + gp - gn))
        return tuple(new)

    accs = lax.fori_loop(
        0, _W, body, tuple(jnp.zeros((_L,), jnp.float32) for _ in range(ngrp)))
    for g in range(ngrp):
        out_v[pl.ds(g * _L, _L)] = accs[g]
    pltpu.sync_copy(out_v, out_hbm.at[pl.ds(base, _BPW)])


def _reshuffle(idx):
    return idx.T.reshape(_W, _NW, _BPW).transpose(1, 0, 2).reshape(_NW, _W * _BPW)


@jax.jit
def _impl(mu_q, sigma_q, pos, neg, mu_table, log_sigma_table):
    mtp = jnp.pad(mu_table, ((0, _VPAD - _VOCAB), (0, 0)))
    lsp = jnp.pad(log_sigma_table, ((0, _VPAD - _VOCAB), (0, 0)))
    bb = _B // _GRID
    h = pl.pallas_call(
        _scores_body,
        grid=(_GRID,),
        in_specs=[
            pl.BlockSpec((bb, _ZD), lambda i: (i, 0)),
            pl.BlockSpec((bb, 1), lambda i: (i, 0)),
            pl.BlockSpec((_ZD, _VPAD), lambda i: (0, 0)),
            pl.BlockSpec((1, _VPAD), lambda i: (0, 0)),
        ],
        out_specs=pl.BlockSpec((bb, _VPAD), lambda i: (i, 0)),
        out_shape=jax.ShapeDtypeStruct((_B, _VPAD), jnp.float32),
    )(mu_q, sigma_q, mtp.T, lsp.T)

    mesh = plsc.VectorSubcoreMesh(core_axis_name="c", subcore_axis_name="s")
    sc = functools.partial(
        pl.kernel,
        mesh=mesh,
        compiler_params=pltpu.CompilerParams(needs_layout_passes=False),
        out_type=jax.ShapeDtypeStruct((_B,), jnp.float32),
        scratch_types=[
            pltpu.VMEM((_BPW, _VPAD), jnp.float32),
            pltpu.VMEM((_W, _BPW), jnp.int32),
            pltpu.VMEM((_BPW,), jnp.float32),
        ],
    )(_sc_hinge_body)
    out = sc(h, (pos | (neg << 7)).T)
    return out.reshape(_B, 1)


def kernel(mu_q, sigma_q, pos_context_words, neg_context_words, mu_table, log_sigma_table):
    return _impl(mu_q, sigma_q, pos_context_words, neg_context_words,
                 mu_table, log_sigma_table)


# speedup vs baseline: 1.0039x; 1.0039x over previous
import functools

import jax
import jax.numpy as jnp
from jax import lax
from jax.experimental import pallas as pl
from jax.experimental.pallas import tpu as pltpu
from jax.experimental.pallas import tpu_sc as plsc

_MARGIN = 5.0
_ZD = 64
_VOCAB = 100
_VPAD = 128
_B = 4096
_W = 50
_NC = 2
_NS = 16
_NW = _NC * _NS
_BPW = _B // _NW
_L = 16
_GRID = 1


def _scores_body(mu_ref, sq_ref, mtT_ref, lsT_ref, out_ref):
    mu = mu_ref[...]
    sq = sq_ref[...]
    mtT = mtT_ref[...]
    ls = lsT_ref[...]
    r = jnp.exp(-ls)
    n = jnp.sum(mtT * mtT, axis=0, keepdims=True)
    m = jnp.sum(mu * mu, axis=1, keepdims=True)
    d = jnp.dot(mu, mtT, preferred_element_type=jnp.float32)
    h = 32.0 * ls + (32.0 * sq + 0.5 * (m + n) - d) * r
    out_ref[...] = h - 32.0 * jnp.log(sq) - 32.0


def _sc_hinge_body(h_hbm, pk_hbm, out_hbm, h_v, pk_v, out_v):
    wid = lax.axis_index("s") * _NC + lax.axis_index("c")
    base = wid * _BPW
    pltpu.sync_copy(h_hbm.at[pl.ds(base, _BPW)], h_v)
    pltpu.sync_copy(pk_hbm.at[:, pl.ds(base, _BPW)], pk_v)
    lane = lax.iota(jnp.int32, _L)
    ngrp = _BPW // _L
    brows = [g * _L + lane for g in range(ngrp)]

    def body(w, accs):
        new = []
        for g in range(ngrp):
            pk = pk_v[w, pl.ds(g * _L, _L)]
            pv = pk & 127
            nv = lax.shift_right_logical(pk, 7)
            gp = plsc.load_gather(h_v, [brows[g], pv])
            gn = plsc.load_gather(h_v, [brows[g], nv])
            new.append(accs[g] + jnp.maximum(0.0, _MARGIN + gp - gn))
        return tuple(new)

    accs = lax.fori_loop(
        0, _W, body, tuple(jnp.zeros((_L,), jnp.float32) for _ in range(ngrp)))
    for g in range(ngrp):
        out_v[pl.ds(g * _L, _L)] = accs[g]
    pltpu.sync_copy(out_v, out_hbm.at[pl.ds(base, _BPW)])


def _reshuffle(idx):
    return idx.T.reshape(_W, _NW, _BPW).transpose(1, 0, 2).reshape(_NW, _W * _BPW)


@jax.jit
def _impl(mu_q, sigma_q, pos, neg, mu_table, log_sigma_table):
    mtp = jnp.pad(mu_table, ((0, _VPAD - _VOCAB), (0, 0)))
    lsp = jnp.pad(log_sigma_table, ((0, _VPAD - _VOCAB), (0, 0)))
    bb = _B // _GRID
    h = pl.pallas_call(
        _scores_body,
        grid=(_GRID,),
        in_specs=[
            pl.BlockSpec((bb, _ZD), lambda i: (i, 0)),
            pl.BlockSpec((bb, 1), lambda i: (i, 0)),
            pl.BlockSpec((_ZD, _VPAD), lambda i: (0, 0)),
            pl.BlockSpec((1, _VPAD), lambda i: (0, 0)),
        ],
        out_specs=pl.BlockSpec((bb, _VPAD), lambda i: (i, 0)),
        out_shape=jax.ShapeDtypeStruct((_B, _VPAD), jnp.float32),
    )(mu_q, sigma_q, mtp.T, lsp.T)

    mesh = plsc.VectorSubcoreMesh(core_axis_name="c", subcore_axis_name="s")
    sc = functools.partial(
        pl.kernel,
        mesh=mesh,
        compiler_params=pltpu.CompilerParams(needs_layout_passes=False),
        out_type=jax.ShapeDtypeStruct((_B,), jnp.float32),
        scratch_types=[
            pltpu.VMEM((_BPW, _VPAD), jnp.float32),
            pltpu.VMEM((_W, _BPW), jnp.int32),
            pltpu.VMEM((_BPW,), jnp.float32),
        ],
    )(_sc_hinge_body)
    out = sc(h, (pos | (neg << 7)).T)
    return out.reshape(_B, 1)


def kernel(mu_q, sigma_q, pos_context_words, neg_context_words, mu_table, log_sigma_table):
    return _impl(mu_q, sigma_q, pos_context_words, neg_context_words,
                 mu_table, log_sigma_table)


# speedup vs baseline: 1.0252x; 1.0212x over previous
import functools

import jax
import jax.numpy as jnp
from jax import lax
from jax.experimental import pallas as pl
from jax.experimental.pallas import tpu as pltpu
from jax.experimental.pallas import tpu_sc as plsc

_MARGIN = 5.0
_ZD = 64
_VOCAB = 100
_VPAD = 128
_B = 4096
_W = 50
_NC = 2
_NS = 16
_NW = _NC * _NS
_BPW = _B // _NW
_L = 16
_GRID = 2


def _scores_body(mu_ref, sq_ref, mtT_ref, lsT_ref, out_ref):
    mu = mu_ref[...]
    sq = sq_ref[...]
    mtT = mtT_ref[...]
    ls = lsT_ref[...]
    r = jnp.exp(-ls)
    n = jnp.sum(mtT * mtT, axis=0, keepdims=True)
    m = jnp.sum(mu * mu, axis=1, keepdims=True)
    d = jnp.dot(mu, mtT, preferred_element_type=jnp.float32)
    h = 32.0 * ls + (32.0 * sq + 0.5 * (m + n) - d) * r
    out_ref[...] = h - 32.0 * jnp.log(sq) - 32.0


def _sc_hinge_body(h_hbm, pk_hbm, out_hbm, h_v, pk_v, out_v):
    wid = lax.axis_index("s") * _NC + lax.axis_index("c")
    base = wid * _BPW
    pltpu.sync_copy(h_hbm.at[pl.ds(base, _BPW)], h_v)
    pltpu.sync_copy(pk_hbm.at[:, pl.ds(base, _BPW)], pk_v)
    lane = lax.iota(jnp.int32, _L)
    ngrp = _BPW // _L
    brows = [g * _L + lane for g in range(ngrp)]

    def body(w, accs):
        new = []
        for g in range(ngrp):
            pk = pk_v[w, pl.ds(g * _L, _L)]
            pv = pk & 127
            nv = lax.shift_right_logical(pk, 7)
            gp = plsc.load_gather(h_v, [brows[g], pv])
            gn = plsc.load_gather(h_v, [brows[g], nv])
            new.append(accs[g] + jnp.maximum(0.0, _MARGIN + gp - gn))
        return tuple(new)

    accs = lax.fori_loop(
        0, _W, body, tuple(jnp.zeros((_L,), jnp.float32) for _ in range(ngrp)))
    for g in range(ngrp):
        out_v[pl.ds(g * _L, _L)] = accs[g]
    pltpu.sync_copy(out_v, out_hbm.at[pl.ds(base, _BPW)])


def _reshuffle(idx):
    return idx.T.reshape(_W, _NW, _BPW).transpose(1, 0, 2).reshape(_NW, _W * _BPW)


@jax.jit
def _impl(mu_q, sigma_q, pos, neg, mu_table, log_sigma_table):
    mtp = jnp.pad(mu_table, ((0, _VPAD - _VOCAB), (0, 0)))
    lsp = jnp.pad(log_sigma_table, ((0, _VPAD - _VOCAB), (0, 0)))
    bb = _B // _GRID
    h = pl.pallas_call(
        _scores_body,
        grid=(_GRID,),
        in_specs=[
            pl.BlockSpec((bb, _ZD), lambda i: (i, 0)),
            pl.BlockSpec((bb, 1), lambda i: (i, 0)),
            pl.BlockSpec((_ZD, _VPAD), lambda i: (0, 0)),
            pl.BlockSpec((1, _VPAD), lambda i: (0, 0)),
        ],
        out_specs=pl.BlockSpec((bb, _VPAD), lambda i: (i, 0)),
        out_shape=jax.ShapeDtypeStruct((_B, _VPAD), jnp.float32),
    )(mu_q, sigma_q, mtp.T, lsp.T)

    mesh = plsc.VectorSubcoreMesh(core_axis_name="c", subcore_axis_name="s")
    sc = functools.partial(
        pl.kernel,
        mesh=mesh,
        compiler_params=pltpu.CompilerParams(
            needs_layout_passes=False,
            disable_bounds_checks=True,
            disable_semaphore_checks=True,
            skip_device_barrier=True,
        ),
        out_type=jax.ShapeDtypeStruct((_B,), jnp.float32),
        scratch_types=[
            pltpu.VMEM((_BPW, _VPAD), jnp.float32),
            pltpu.VMEM((_W, _BPW), jnp.int32),
            pltpu.VMEM((_BPW,), jnp.float32),
        ],
    )(_sc_hinge_body)
    out = sc(h, (pos | (neg << 7)).T)
    return out.reshape(_B, 1)


def kernel(mu_q, sigma_q, pos_context_words, neg_context_words, mu_table, log_sigma_table):
    return _impl(mu_q, sigma_q, pos_context_words, neg_context_words,
                 mu_table, log_sigma_table)
